# baseline (speedup 1.0000x reference)
import functools

import jax
import jax.numpy as jnp
import numpy as np
from jax import lax
from jax.experimental import pallas as pl
from jax.experimental.pallas import tpu as pltpu
from jax.experimental.pallas import tpu_sc as plsc

_LEFT_HAND = np.arange(468, 489)
_LEFT_POSE = np.array([502, 504, 506, 508, 510])
_LIPS = np.array([
    61, 185, 40, 39, 37, 0, 267, 269, 270, 409, 291, 146, 91, 181, 84, 17,
    314, 405, 321, 375, 78, 191, 80, 81, 82, 13, 312, 311, 310, 415, 95, 88,
    178, 87, 14, 317, 402, 318, 324, 308,
])
_LM_LEFT = np.concatenate((_LIPS, _LEFT_HAND, _LEFT_POSE))
_RIGHT_HAND = np.arange(522, 543)
_RIGHT_POSE = np.array([503, 505, 507, 509, 511])
_LM_RIGHT = np.concatenate((_LIPS, _RIGHT_HAND, _RIGHT_POSE))

N_FRAMES = 4096
N_LM = 543
ROW_W = 2 * N_LM
N_OUT_LM = 66
OUT_F = 64
NS = 16
L = 16
FPT = N_FRAMES // NS
NG = FPT // L
NGATHER = 9
OUT_W = NGATHER * L
N128 = N_FRAMES * ROW_W // 128
RPF = 10
CHF = 8
NCH = OUT_F // CHF


def _flat_idx(lm):
    fi = np.stack([2 * lm, 2 * lm + 1], axis=1).reshape(-1)
    return np.pad(fi, (0, OUT_W - fi.size)).astype(np.int32)


_LIDX_L = _flat_idx(_LM_LEFT)
_LIDX_R = _flat_idx(_LM_RIGHT)

_MUL_R = np.ones(OUT_W, np.float32)
_ADD_R = np.zeros(OUT_W, np.float32)
for _l in range(40, N_OUT_LM):
    _MUL_R[2 * _l] = -1.0
    _ADD_R[2 * _l] = 1.0


@functools.cache
def _build_sc_kernel():
    mesh = plsc.VectorSubcoreMesh(
        core_axis_name="c", subcore_axis_name="s", num_cores=2,
        num_subcores=NS,
    )

    @functools.partial(
        pl.kernel,
        out_type=[
            jax.ShapeDtypeStruct((OUT_F * OUT_W,), jnp.float32),
            jax.ShapeDtypeStruct((N_FRAMES,), jnp.int32),
        ],
        mesh=mesh,
        compiler_params=pltpu.CompilerParams(
            needs_layout_passes=False, use_tc_tiling_on_sc=False),
        scratch_types=[
            pltpu.VMEM((NG, 96, L), jnp.float32),
            pltpu.VMEM((FPT,), jnp.int32),
            pltpu.VMEM((FPT,), jnp.int32),
            pltpu.VMEM((L,), jnp.int32),
            pltpu.VMEM((L,), jnp.int32),
            pltpu.VMEM_SHARED((N_FRAMES,), jnp.int32),
            pltpu.VMEM_SHARED((N_FRAMES,), jnp.int32),
            pltpu.VMEM_SHARED((NS, L), jnp.int32),
            pltpu.VMEM_SHARED((NS, L), jnp.int32),
            pltpu.VMEM((N_FRAMES,), jnp.int32),
            pltpu.VMEM((N_FRAMES,), jnp.int32),
            pltpu.VMEM((NS, L), jnp.int32),
            pltpu.VMEM((NS, L), jnp.int32),
            pltpu.VMEM((N_FRAMES,), jnp.int32),
            pltpu.VMEM((CHF * RPF,), jnp.int32),
            pltpu.VMEM((CHF * RPF, 128), jnp.float32),
            pltpu.VMEM((OUT_F * OUT_W,), jnp.float32),
            pltpu.VMEM((OUT_W,), jnp.int32),
            pltpu.VMEM((OUT_W,), jnp.int32),
            pltpu.VMEM((OUT_W,), jnp.float32),
            pltpu.VMEM((OUT_W,), jnp.float32),
            pltpu.VMEM((OUT_W,), jnp.int32),
            pltpu.VMEM((OUT_W,), jnp.float32),
            pltpu.VMEM((OUT_W,), jnp.float32),
            pltpu.SemaphoreType.DMA,
        ],
    )
    def _sc_kernel(
        hands_hbm, x128_hbm, lidxL_hbm, lidxR_hbm, mulR_hbm, addR_hbm,
        out1_hbm, oidx_hbm,
        hbuf, cntL_l, cntR_l, stageL, stageR,
        sh_cntL, sh_cntR, sh_totL, sh_totR,
        cntL_all, cntR_all, totL_all, totR_all,
        idx_buf, idx80, rows8, obuf,
        lidxL_v, lidxR_v, mulR_v, addR_v, lidx_sel, mul_sel, add_sel,
        sem,
    ):
        c = lax.axis_index("c")
        s = lax.axis_index("s")

        @pl.when(c == 0)
        def _core0():
            pltpu.sync_copy(hands_hbm.at[s], hbuf)

            def group(g, tots):
                tL, tR = tots

                def cnt(lo, hi):
                    def body(e, a):
                        v = hbuf[g, e, :]
                        return a + (v != v).astype(jnp.int32)

                    return lax.fori_loop(
                        lo, hi, body, jnp.zeros((L,), jnp.int32))

                aL = cnt(0, 48)
                aR = cnt(48, 96)
                cntL_l[pl.ds(g * L, L)] = aL
                cntR_l[pl.ds(g * L, L)] = aR
                return (tL + aL, tR + aR)

            totL, totR = lax.fori_loop(
                0, NG, group,
                (jnp.zeros((L,), jnp.int32), jnp.zeros((L,), jnp.int32)),
            )
            stageL[...] = jnp.full((L,), jnp.sum(totL), jnp.int32)
            stageR[...] = jnp.full((L,), jnp.sum(totR), jnp.int32)
            pltpu.sync_copy(cntL_l, sh_cntL.at[pl.ds(s * FPT, FPT)])
            pltpu.sync_copy(cntR_l, sh_cntR.at[pl.ds(s * FPT, FPT)])
            pltpu.sync_copy(stageL, sh_totL.at[s])
            pltpu.sync_copy(stageR, sh_totR.at[s])
            plsc.subcore_barrier()

            @pl.when(s == 0)
            def _tile0():
                pltpu.sync_copy(sh_totL, totL_all)
                pltpu.sync_copy(sh_totR, totR_all)

                def tot_body(i, a):
                    aL, aR = a
                    return (aL + totL_all[i, :], aR + totR_all[i, :])

                accL, accR = lax.fori_loop(
                    0, NS, tot_body,
                    (jnp.zeros((L,), jnp.int32), jnp.zeros((L,), jnp.int32)),
                )
                ld = accL <= accR

                pltpu.sync_copy(sh_cntL, cntL_all)
                pltpu.sync_copy(sh_cntR, cntR_all)

                def zero(i, carry):
                    idx_buf[pl.ds(i * L, L)] = jnp.zeros((L,), jnp.int32)
                    return carry

                lax.fori_loop(0, N_FRAMES // L, zero, 0)

                def comp(g, carry):
                    cl = cntL_all[pl.ds(g * L, L)]
                    cr = cntR_all[pl.ds(g * L, L)]
                    cnt = jnp.where(ld, cl, cr)
                    m = cnt < 48
                    mi = m.astype(jnp.int32)
                    pos = carry + plsc.cumsum(mi) - mi
                    fid = g * L + lax.iota(jnp.int32, L)
                    plsc.store_scatter(idx_buf, [pos], fid, mask=m)
                    return carry + plsc.all_reduce_population_count(m)

                lax.fori_loop(
                    0, N_FRAMES // L, comp, jnp.zeros((L,), jnp.int32))
                pltpu.sync_copy(idx_buf, oidx_hbm)

                pltpu.sync_copy(lidxL_hbm, lidxL_v)
                pltpu.sync_copy(lidxR_hbm, lidxR_v)
                pltpu.sync_copy(mulR_hbm, mulR_v)
                pltpu.sync_copy(addR_hbm, addR_v)
                for v in range(NGATHER):
                    sl = pl.ds(v * L, L)
                    lidx_sel[sl] = jnp.where(ld, lidxL_v[sl], lidxR_v[sl])
                    mul_sel[sl] = jnp.where(
                        ld, jnp.full((L,), 1.0, jnp.float32), mulR_v[sl])
                    add_sel[sl] = jnp.where(
                        ld, jnp.full((L,), 0.0, jnp.float32), addR_v[sl])

                lane = lax.iota(jnp.int32, L)
                lmask = lane < CHF

                def chunk(ch, carry):
                    fidx = idx_buf[pl.ds(ch * CHF, L)]
                    fbase = fidx * ROW_W
                    r0 = lax.shift_right_logical(fbase, 7)
                    offs = jnp.bitwise_and(fbase, 127)
                    for k in range(RPF):
                        rk = jnp.minimum(r0 + k, N128 - 1)
                        plsc.store_scatter(
                            idx80, [lane * RPF + k], rk, mask=lmask)
                    pltpu.async_copy(x128_hbm.at[idx80], rows8, sem).wait()
                    for f in range(CHF):
                        off = offs[jnp.full((L,), f, jnp.int32)]
                        for v in range(NGATHER):
                            sl = pl.ds(v * L, L)
                            colraw = off + lidx_sel[sl]
                            rsel = f * RPF + lax.shift_right_logical(colraw, 7)
                            csel = jnp.bitwise_and(colraw, 127)
                            vals = plsc.load_gather(rows8, [rsel, csel])
                            t = vals * mul_sel[sl] + add_sel[sl]
                            t = jnp.where(vals != vals, jnp.float32(0.0), t)
                            obuf[pl.ds(((ch * CHF + f) * NGATHER + v) * L, L)] = t
                    return carry

                lax.fori_loop(0, NCH, chunk, 0)
                pltpu.sync_copy(obuf, out1_hbm)

    return _sc_kernel


def kernel(x):
    xf = x.reshape(N_FRAMES, ROW_W)
    xl = lax.slice(xf, (0, 2 * 468), (N_FRAMES, 2 * 489))
    xr = lax.slice(xf, (0, 2 * 522), (N_FRAMES, 2 * 543))
    nanpad = jnp.full((N_FRAMES, 6), jnp.nan, jnp.float32)
    hands = jnp.concatenate([xl, nanpad, xr, nanpad], axis=1)
    h = hands.reshape(NS, NG, L, 96).transpose(0, 1, 3, 2)
    x128 = xf.reshape(N128, 128)
    out1, oidx = _build_sc_kernel()(
        h, x128,
        jnp.asarray(_LIDX_L), jnp.asarray(_LIDX_R),
        jnp.asarray(_MUL_R), jnp.asarray(_ADD_R),
    )
    x1 = out1.reshape(OUT_F, OUT_W)[:, : 2 * N_OUT_LM].reshape(
        OUT_F, N_OUT_LM, 2)
    return (x1, oidx)


# speedup vs baseline: 1.0076x; 1.0076x over previous
import functools

import jax
import jax.numpy as jnp
import numpy as np
from jax import lax
from jax.experimental import pallas as pl
from jax.experimental.pallas import tpu as pltpu
from jax.experimental.pallas import tpu_sc as plsc

_LEFT_HAND = np.arange(468, 489)
_LEFT_POSE = np.array([502, 504, 506, 508, 510])
_LIPS = np.array([
    61, 185, 40, 39, 37, 0, 267, 269, 270, 409, 291, 146, 91, 181, 84, 17,
    314, 405, 321, 375, 78, 191, 80, 81, 82, 13, 312, 311, 310, 415, 95, 88,
    178, 87, 14, 317, 402, 318, 324, 308,
])
_LM_LEFT = np.concatenate((_LIPS, _LEFT_HAND, _LEFT_POSE))
_RIGHT_HAND = np.arange(522, 543)
_RIGHT_POSE = np.array([503, 505, 507, 509, 511])
_LM_RIGHT = np.concatenate((_LIPS, _RIGHT_HAND, _RIGHT_POSE))

N_FRAMES = 4096
N_LM = 543
ROW_W = 2 * N_LM
N_OUT_LM = 66
OUT_F = 64
NS = 16
L = 16
FPT = N_FRAMES // NS
NG = FPT // L
NGATHER = 9
OUT_W = NGATHER * L
N128 = N_FRAMES * ROW_W // 128
RPF = 10
CHF = 8
NCH = OUT_F // CHF


def _flat_idx(lm):
    fi = np.stack([2 * lm, 2 * lm + 1], axis=1).reshape(-1)
    return np.pad(fi, (0, OUT_W - fi.size)).astype(np.int32)


_LIDX_L = _flat_idx(_LM_LEFT)
_LIDX_R = _flat_idx(_LM_RIGHT)

_MUL_R = np.ones(OUT_W, np.float32)
_ADD_R = np.zeros(OUT_W, np.float32)
for _l in range(40, N_OUT_LM):
    _MUL_R[2 * _l] = -1.0
    _ADD_R[2 * _l] = 1.0


@functools.cache
def _build_sc_kernel():
    mesh = plsc.VectorSubcoreMesh(
        core_axis_name="c", subcore_axis_name="s", num_cores=2,
        num_subcores=NS,
    )

    @functools.partial(
        pl.kernel,
        out_type=[
            jax.ShapeDtypeStruct((OUT_F * OUT_W,), jnp.float32),
            jax.ShapeDtypeStruct((N_FRAMES,), jnp.int32),
        ],
        mesh=mesh,
        compiler_params=pltpu.CompilerParams(
            needs_layout_passes=False, use_tc_tiling_on_sc=False),
        scratch_types=[
            pltpu.VMEM((FPT, 84), jnp.float32),
            pltpu.VMEM((FPT,), jnp.int32),
            pltpu.VMEM((FPT,), jnp.int32),
            pltpu.VMEM((L,), jnp.int32),
            pltpu.VMEM((L,), jnp.int32),
            pltpu.VMEM_SHARED((N_FRAMES,), jnp.int32),
            pltpu.VMEM_SHARED((N_FRAMES,), jnp.int32),
            pltpu.VMEM_SHARED((NS, L), jnp.int32),
            pltpu.VMEM_SHARED((NS, L), jnp.int32),
            pltpu.VMEM((N_FRAMES,), jnp.int32),
            pltpu.VMEM((N_FRAMES,), jnp.int32),
            pltpu.VMEM((NS, L), jnp.int32),
            pltpu.VMEM((NS, L), jnp.int32),
            pltpu.VMEM((N_FRAMES,), jnp.int32),
            pltpu.VMEM((CHF * RPF,), jnp.int32),
            pltpu.VMEM((CHF * RPF, 128), jnp.float32),
            pltpu.VMEM((OUT_F * OUT_W,), jnp.float32),
            pltpu.VMEM((OUT_W,), jnp.int32),
            pltpu.VMEM((OUT_W,), jnp.int32),
            pltpu.VMEM((OUT_W,), jnp.float32),
            pltpu.VMEM((OUT_W,), jnp.float32),
            pltpu.VMEM((OUT_W,), jnp.int32),
            pltpu.VMEM((OUT_W,), jnp.float32),
            pltpu.VMEM((OUT_W,), jnp.float32),
            pltpu.SemaphoreType.DMA,
        ],
    )
    def _sc_kernel(
        hands_hbm, x128_hbm, lidxL_hbm, lidxR_hbm, mulR_hbm, addR_hbm,
        out1_hbm, oidx_hbm,
        hbuf, cntL_l, cntR_l, stageL, stageR,
        sh_cntL, sh_cntR, sh_totL, sh_totR,
        cntL_all, cntR_all, totL_all, totR_all,
        idx_buf, idx80, rows8, obuf,
        lidxL_v, lidxR_v, mulR_v, addR_v, lidx_sel, mul_sel, add_sel,
        sem,
    ):
        c = lax.axis_index("c")
        s = lax.axis_index("s")

        @pl.when(c == 0)
        def _core0():
            pltpu.sync_copy(hands_hbm.at[pl.ds(s * FPT, FPT)], hbuf)

            def group(g, tots):
                tL, tR = tots
                fidx = g * L + lax.iota(jnp.int32, L)

                def cnt(lo, hi):
                    def body(e, a):
                        ev = jnp.full((L,), e, jnp.int32)
                        v = plsc.load_gather(hbuf, [fidx, ev])
                        return a + (v != v).astype(jnp.int32)

                    return lax.fori_loop(
                        lo, hi, body, jnp.zeros((L,), jnp.int32))

                aL = cnt(0, 42)
                aR = cnt(42, 84)
                cntL_l[pl.ds(g * L, L)] = aL
                cntR_l[pl.ds(g * L, L)] = aR
                return (tL + aL, tR + aR)

            totL, totR = lax.fori_loop(
                0, NG, group,
                (jnp.zeros((L,), jnp.int32), jnp.zeros((L,), jnp.int32)),
            )
            stageL[...] = jnp.full((L,), jnp.sum(totL), jnp.int32)
            stageR[...] = jnp.full((L,), jnp.sum(totR), jnp.int32)
            pltpu.sync_copy(cntL_l, sh_cntL.at[pl.ds(s * FPT, FPT)])
            pltpu.sync_copy(cntR_l, sh_cntR.at[pl.ds(s * FPT, FPT)])
            pltpu.sync_copy(stageL, sh_totL.at[s])
            pltpu.sync_copy(stageR, sh_totR.at[s])
            plsc.subcore_barrier()

            @pl.when(s == 0)
            def _tile0():
                pltpu.sync_copy(sh_totL, totL_all)
                pltpu.sync_copy(sh_totR, totR_all)

                def tot_body(i, a):
                    aL, aR = a
                    return (aL + totL_all[i, :], aR + totR_all[i, :])

                accL, accR = lax.fori_loop(
                    0, NS, tot_body,
                    (jnp.zeros((L,), jnp.int32), jnp.zeros((L,), jnp.int32)),
                )
                ld = accL <= accR

                pltpu.sync_copy(sh_cntL, cntL_all)
                pltpu.sync_copy(sh_cntR, cntR_all)

                def zero(i, carry):
                    idx_buf[pl.ds(i * L, L)] = jnp.zeros((L,), jnp.int32)
                    return carry

                lax.fori_loop(0, N_FRAMES // L, zero, 0)

                def comp(g, carry):
                    cl = cntL_all[pl.ds(g * L, L)]
                    cr = cntR_all[pl.ds(g * L, L)]
                    cnt = jnp.where(ld, cl, cr)
                    m = cnt < 42
                    mi = m.astype(jnp.int32)
                    pos = carry + plsc.cumsum(mi) - mi
                    fid = g * L + lax.iota(jnp.int32, L)
                    plsc.store_scatter(idx_buf, [pos], fid, mask=m)
                    return carry + plsc.all_reduce_population_count(m)

                lax.fori_loop(
                    0, N_FRAMES // L, comp, jnp.zeros((L,), jnp.int32))
                pltpu.sync_copy(idx_buf, oidx_hbm)

                pltpu.sync_copy(lidxL_hbm, lidxL_v)
                pltpu.sync_copy(lidxR_hbm, lidxR_v)
                pltpu.sync_copy(mulR_hbm, mulR_v)
                pltpu.sync_copy(addR_hbm, addR_v)
                for v in range(NGATHER):
                    sl = pl.ds(v * L, L)
                    lidx_sel[sl] = jnp.where(ld, lidxL_v[sl], lidxR_v[sl])
                    mul_sel[sl] = jnp.where(
                        ld, jnp.full((L,), 1.0, jnp.float32), mulR_v[sl])
                    add_sel[sl] = jnp.where(
                        ld, jnp.full((L,), 0.0, jnp.float32), addR_v[sl])

                lane = lax.iota(jnp.int32, L)
                lmask = lane < CHF

                def chunk(ch, carry):
                    fidx = idx_buf[pl.ds(ch * CHF, L)]
                    fbase = fidx * ROW_W
                    r0 = lax.shift_right_logical(fbase, 7)
                    offs = jnp.bitwise_and(fbase, 127)
                    for k in range(RPF):
                        rk = jnp.minimum(r0 + k, N128 - 1)
                        plsc.store_scatter(
                            idx80, [lane * RPF + k], rk, mask=lmask)
                    pltpu.async_copy(x128_hbm.at[idx80], rows8, sem).wait()
                    for f in range(CHF):
                        off = offs[jnp.full((L,), f, jnp.int32)]
                        for v in range(NGATHER):
                            sl = pl.ds(v * L, L)
                            colraw = off + lidx_sel[sl]
                            rsel = f * RPF + lax.shift_right_logical(colraw, 7)
                            csel = jnp.bitwise_and(colraw, 127)
                            vals = plsc.load_gather(rows8, [rsel, csel])
                            t = vals * mul_sel[sl] + add_sel[sl]
                            t = jnp.where(vals != vals, jnp.float32(0.0), t)
                            obuf[pl.ds(((ch * CHF + f) * NGATHER + v) * L, L)] = t
                    return carry

                lax.fori_loop(0, NCH, chunk, 0)
                pltpu.sync_copy(obuf, out1_hbm)

    return _sc_kernel


def kernel(x):
    xf = x.reshape(N_FRAMES, ROW_W)
    xl = lax.slice(xf, (0, 2 * 468), (N_FRAMES, 2 * 489))
    xr = lax.slice(xf, (0, 2 * 522), (N_FRAMES, 2 * 543))
    hands = jnp.concatenate([xl, xr], axis=1)
    x128 = xf.reshape(N128, 128)
    out1, oidx = _build_sc_kernel()(
        hands, x128,
        jnp.asarray(_LIDX_L), jnp.asarray(_LIDX_R),
        jnp.asarray(_MUL_R), jnp.asarray(_ADD_R),
    )
    x1 = out1.reshape(OUT_F, OUT_W)[:, : 2 * N_OUT_LM].reshape(
        OUT_F, N_OUT_LM, 2)
    return (x1, oidx)


# speedup vs baseline: 1.0117x; 1.0040x over previous
import functools

import jax
import jax.numpy as jnp
import numpy as np
from jax import lax
from jax.experimental import pallas as pl
from jax.experimental.pallas import tpu as pltpu
from jax.experimental.pallas import tpu_sc as plsc

_LEFT_HAND = np.arange(468, 489)
_LEFT_POSE = np.array([502, 504, 506, 508, 510])
_LIPS = np.array([
    61, 185, 40, 39, 37, 0, 267, 269, 270, 409, 291, 146, 91, 181, 84, 17,
    314, 405, 321, 375, 78, 191, 80, 81, 82, 13, 312, 311, 310, 415, 95, 88,
    178, 87, 14, 317, 402, 318, 324, 308,
])
_LM_LEFT = np.concatenate((_LIPS, _LEFT_HAND, _LEFT_POSE))
_RIGHT_HAND = np.arange(522, 543)
_RIGHT_POSE = np.array([503, 505, 507, 509, 511])
_LM_RIGHT = np.concatenate((_LIPS, _RIGHT_HAND, _RIGHT_POSE))

N_FRAMES = 4096
N_LM = 543
ROW_W = 2 * N_LM
N_OUT_LM = 66
OUT_F = 64
NS = 16
L = 16
FPT = N_FRAMES // NS
NG = FPT // L
NGATHER = 9
OUT_W = NGATHER * L
N128 = N_FRAMES * ROW_W // 128
RPF = 10
CHF = 8
NCH = OUT_F // CHF


def _flat_idx(lm):
    fi = np.stack([2 * lm, 2 * lm + 1], axis=1).reshape(-1)
    return np.pad(fi, (0, OUT_W - fi.size)).astype(np.int32)


_LIDX_L = _flat_idx(_LM_LEFT)
_LIDX_R = _flat_idx(_LM_RIGHT)

_MUL_R = np.ones(OUT_W, np.float32)
_ADD_R = np.zeros(OUT_W, np.float32)
for _l in range(40, N_OUT_LM):
    _MUL_R[2 * _l] = -1.0
    _ADD_R[2 * _l] = 1.0


@functools.cache
def _build_sc_kernel():
    mesh = plsc.VectorSubcoreMesh(
        core_axis_name="c", subcore_axis_name="s", num_cores=2,
        num_subcores=NS,
    )

    @functools.partial(
        pl.kernel,
        out_type=[
            jax.ShapeDtypeStruct((OUT_F * OUT_W,), jnp.float32),
            jax.ShapeDtypeStruct((N_FRAMES,), jnp.int32),
        ],
        mesh=mesh,
        compiler_params=pltpu.CompilerParams(
            needs_layout_passes=False, use_tc_tiling_on_sc=False),
        scratch_types=[
            pltpu.VMEM((543, 128), jnp.float32),
            pltpu.VMEM((FPT,), jnp.int32),
            pltpu.VMEM((FPT,), jnp.int32),
            pltpu.VMEM((L,), jnp.int32),
            pltpu.VMEM((L,), jnp.int32),
            pltpu.VMEM_SHARED((N_FRAMES,), jnp.int32),
            pltpu.VMEM_SHARED((N_FRAMES,), jnp.int32),
            pltpu.VMEM_SHARED((NS, L), jnp.int32),
            pltpu.VMEM_SHARED((NS, L), jnp.int32),
            pltpu.VMEM((N_FRAMES,), jnp.int32),
            pltpu.VMEM((N_FRAMES,), jnp.int32),
            pltpu.VMEM((NS, L), jnp.int32),
            pltpu.VMEM((NS, L), jnp.int32),
            pltpu.VMEM((N_FRAMES,), jnp.int32),
            pltpu.VMEM((CHF * RPF,), jnp.int32),
            pltpu.VMEM((CHF * RPF, 128), jnp.float32),
            pltpu.VMEM((OUT_F * OUT_W,), jnp.float32),
            pltpu.VMEM((OUT_W,), jnp.int32),
            pltpu.VMEM((OUT_W,), jnp.int32),
            pltpu.VMEM((OUT_W,), jnp.float32),
            pltpu.VMEM((OUT_W,), jnp.float32),
            pltpu.VMEM((OUT_W,), jnp.int32),
            pltpu.VMEM((OUT_W,), jnp.float32),
            pltpu.VMEM((OUT_W,), jnp.float32),
            pltpu.SemaphoreType.DMA,
        ],
    )
    def _sc_kernel(
        x128_hbm, lidxL_hbm, lidxR_hbm, mulR_hbm, addR_hbm,
        out1_hbm, oidx_hbm,
        rowbuf, cntL_l, cntR_l, stageL, stageR,
        sh_cntL, sh_cntR, sh_totL, sh_totR,
        cntL_all, cntR_all, totL_all, totR_all,
        idx_buf, idx80, rows8, obuf,
        lidxL_v, lidxR_v, mulR_v, addR_v, lidx_sel, mul_sel, add_sel,
        sem,
    ):
        c = lax.axis_index("c")
        s = lax.axis_index("s")

        @pl.when(c == 0)
        def _core0():
            CF = FPT // 4
            CR = CF * ROW_W // 128

            def p1chunk(ch, tots0):
                pltpu.sync_copy(
                    x128_hbm.at[pl.ds((s * 4 + ch) * CR, CR)], rowbuf)

                def group(g, tots):
                    tL, tR = tots
                    fbase = (g * L + lax.iota(jnp.int32, L)) * ROW_W

                    def cnt(base):
                        def body(e, a):
                            flat = fbase + (base + e)
                            rsel = lax.shift_right_logical(flat, 7)
                            csel = jnp.bitwise_and(flat, 127)
                            v = plsc.load_gather(rowbuf, [rsel, csel])
                            return a + (v != v).astype(jnp.int32)

                        return lax.fori_loop(
                            0, 42, body, jnp.zeros((L,), jnp.int32))

                    aL = cnt(2 * 468)
                    aR = cnt(2 * 522)
                    cntL_l[pl.ds(ch * CF + g * L, L)] = aL
                    cntR_l[pl.ds(ch * CF + g * L, L)] = aR
                    return (tL + aL, tR + aR)

                return lax.fori_loop(0, CF // L, group, tots0)

            totL, totR = lax.fori_loop(
                0, 4, p1chunk,
                (jnp.zeros((L,), jnp.int32), jnp.zeros((L,), jnp.int32)),
            )
            stageL[...] = jnp.full((L,), jnp.sum(totL), jnp.int32)
            stageR[...] = jnp.full((L,), jnp.sum(totR), jnp.int32)
            pltpu.sync_copy(cntL_l, sh_cntL.at[pl.ds(s * FPT, FPT)])
            pltpu.sync_copy(cntR_l, sh_cntR.at[pl.ds(s * FPT, FPT)])
            pltpu.sync_copy(stageL, sh_totL.at[s])
            pltpu.sync_copy(stageR, sh_totR.at[s])
            plsc.subcore_barrier()

            @pl.when(s == 0)
            def _tile0():
                pltpu.sync_copy(sh_totL, totL_all)
                pltpu.sync_copy(sh_totR, totR_all)

                def tot_body(i, a):
                    aL, aR = a
                    return (aL + totL_all[i, :], aR + totR_all[i, :])

                accL, accR = lax.fori_loop(
                    0, NS, tot_body,
                    (jnp.zeros((L,), jnp.int32), jnp.zeros((L,), jnp.int32)),
                )
                ld = accL <= accR

                pltpu.sync_copy(sh_cntL, cntL_all)
                pltpu.sync_copy(sh_cntR, cntR_all)

                def zero(i, carry):
                    idx_buf[pl.ds(i * L, L)] = jnp.zeros((L,), jnp.int32)
                    return carry

                lax.fori_loop(0, N_FRAMES // L, zero, 0)

                def comp(g, carry):
                    cl = cntL_all[pl.ds(g * L, L)]
                    cr = cntR_all[pl.ds(g * L, L)]
                    cnt = jnp.where(ld, cl, cr)
                    m = cnt < 42
                    mi = m.astype(jnp.int32)
                    pos = carry + plsc.cumsum(mi) - mi
                    fid = g * L + lax.iota(jnp.int32, L)
                    plsc.store_scatter(idx_buf, [pos], fid, mask=m)
                    return carry + plsc.all_reduce_population_count(m)

                lax.fori_loop(
                    0, N_FRAMES // L, comp, jnp.zeros((L,), jnp.int32))
                pltpu.sync_copy(idx_buf, oidx_hbm)

                pltpu.sync_copy(lidxL_hbm, lidxL_v)
                pltpu.sync_copy(lidxR_hbm, lidxR_v)
                pltpu.sync_copy(mulR_hbm, mulR_v)
                pltpu.sync_copy(addR_hbm, addR_v)
                for v in range(NGATHER):
                    sl = pl.ds(v * L, L)
                    lidx_sel[sl] = jnp.where(ld, lidxL_v[sl], lidxR_v[sl])
                    mul_sel[sl] = jnp.where(
                        ld, jnp.full((L,), 1.0, jnp.float32), mulR_v[sl])
                    add_sel[sl] = jnp.where(
                        ld, jnp.full((L,), 0.0, jnp.float32), addR_v[sl])

                lane = lax.iota(jnp.int32, L)
                lmask = lane < CHF

                def chunk(ch, carry):
                    fidx = idx_buf[pl.ds(ch * CHF, L)]
                    fbase = fidx * ROW_W
                    r0 = lax.shift_right_logical(fbase, 7)
                    offs = jnp.bitwise_and(fbase, 127)
                    for k in range(RPF):
                        rk = jnp.minimum(r0 + k, N128 - 1)
                        plsc.store_scatter(
                            idx80, [lane * RPF + k], rk, mask=lmask)
                    pltpu.async_copy(x128_hbm.at[idx80], rows8, sem).wait()
                    for f in range(CHF):
                        off = offs[jnp.full((L,), f, jnp.int32)]
                        for v in range(NGATHER):
                            sl = pl.ds(v * L, L)
                            colraw = off + lidx_sel[sl]
                            rsel = f * RPF + lax.shift_right_logical(colraw, 7)
                            csel = jnp.bitwise_and(colraw, 127)
                            vals = plsc.load_gather(rows8, [rsel, csel])
                            t = vals * mul_sel[sl] + add_sel[sl]
                            t = jnp.where(vals != vals, jnp.float32(0.0), t)
                            obuf[pl.ds(((ch * CHF + f) * NGATHER + v) * L, L)] = t
                    return carry

                lax.fori_loop(0, NCH, chunk, 0)
                pltpu.sync_copy(obuf, out1_hbm)

    return _sc_kernel


def kernel(x):
    x128 = x.reshape(N128, 128)
    out1, oidx = _build_sc_kernel()(
        x128,
        jnp.asarray(_LIDX_L), jnp.asarray(_LIDX_R),
        jnp.asarray(_MUL_R), jnp.asarray(_ADD_R),
    )
    x1 = out1.reshape(OUT_F, OUT_W)[:, : 2 * N_OUT_LM].reshape(
        OUT_F, N_OUT_LM, 2)
    return (x1, oidx)
